# baseline (speedup 1.0000x reference)
import functools

import jax
import jax.numpy as jnp
from jax.experimental import pallas as pl
from jax.experimental.pallas import tpu as pltpu

D_MODEL = 1024
N_HEADS = 16
D_HEAD = D_MODEL // N_HEADS
HP = 128
EPS_LN = 1e-5
NEG = -1e30
SCALE = 1.0 / (D_HEAD ** 0.5)


def _rowstats(x):
    mu = jnp.mean(x, axis=-1, keepdims=True)
    xc = x - mu
    var = jnp.mean(xc * xc, axis=-1, keepdims=True)
    return xc, jax.lax.rsqrt(var + EPS_LN)


def _dot(a, b, dims):
    return jax.lax.dot_general(a, b, (dims, ((), ())),
                               preferred_element_type=jnp.float32)


def _prologue_body(q_ref, nqg_ref, nqb_ref, wq_ref, bq_ref, wk_ref, bk_ref,
                   nkg_ref, nkb_ref, wv_ref, nvg_ref, nvb_ref, g_ref, b_ref,
                   wall_ref, biasp_ref, consts_ref, wvgg_ref, vrows_ref):
    D = D_MODEL
    g = g_ref[...]
    b = b_ref[...]
    ones = jnp.ones((1, D), jnp.float32)

    qc, qr = _rowstats(q_ref[...])
    qn = qc * qr * nqg_ref[...] + nqb_ref[...]
    qs = (_dot(qn, wq_ref[...], ((1,), (1,))) + bq_ref[...]) * SCALE

    lane = jax.lax.broadcasted_iota(jnp.int32, (HP, D), 1)
    sub = jax.lax.broadcasted_iota(jnp.int32, (HP, D), 0)
    ohT = jnp.where(sub == lane // D_HEAD, 1.0, 0.0)
    wsT = _dot(ohT * qs, wk_ref[...], ((1,), (0,)))

    part1 = _dot(qs * bk_ref[...], ohT, ((1,), (1,)))
    part2 = _dot(nkb_ref[...], wsT, ((1,), (1,)))
    hlane = jax.lax.broadcasted_iota(jnp.int32, (1, HP), 1)
    biasp_ref[...] = part1 + part2 + jnp.where(hlane < N_HEADS, 0.0, NEG)

    wsTk = wsT * nkg_ref[...]
    cb = _dot(b, wsTk, ((1,), (1,)))
    c1 = _dot(ones, wsTk, ((1,), (1,)))
    cg = _dot(g, wsTk, ((1,), (1,)))
    mg = jnp.mean(g, axis=-1, keepdims=True)
    mb = jnp.mean(b, axis=-1, keepdims=True)
    mg2 = jnp.mean(g * g, axis=-1, keepdims=True)
    mgb = jnp.mean(g * b, axis=-1, keepdims=True)
    mb2 = jnp.mean(b * b, axis=-1, keepdims=True)
    sl = hlane
    mrow = (jnp.where(sl == 0, mg, 0.0) + jnp.where(sl == 1, mb, 0.0)
            + jnp.where(sl == 2, mg2, 0.0) + jnp.where(sl == 3, mgb, 0.0)
            + jnp.where(sl == 4, mb2, 0.0))
    consts_ref[...] = jnp.concatenate(
        [cb, -c1, -cg, mrow, jnp.zeros((4, HP), jnp.float32)], axis=0)

    statT = (jnp.where(sub == N_HEADS, 1.0 / D, 0.0)
             + jnp.where(sub == N_HEADS + 1, g / D, 0.0)
             + jnp.where(sub == N_HEADS + 2, g * g / D, 0.0)
             + jnp.where(sub == N_HEADS + 3, g * b / D, 0.0))
    wallT = jnp.where(sub < N_HEADS, wsTk * g, statT).astype(jnp.bfloat16)
    ii = jax.lax.broadcasted_iota(jnp.int32, (D, D), 0)
    jj = jax.lax.broadcasted_iota(jnp.int32, (D, D), 1)
    eyeb = jnp.where(ii == jj, 1.0, 0.0).astype(jnp.bfloat16)
    wall_ref[...] = _dot(eyeb, wallT, ((1,), (1,))).astype(jnp.bfloat16)

    wvg = wv_ref[...] * nvg_ref[...]
    wvgg_ref[...] = (wvg * g).astype(jnp.bfloat16)
    vr0 = _dot(nvb_ref[...], wv_ref[...], ((1,), (1,)))
    vr1 = _dot(b, wvg, ((1,), (1,)))
    vr2 = _dot(ones, wvg, ((1,), (1,)))
    vr3 = _dot(g, wvg, ((1,), (1,)))
    vrows_ref[...] = jnp.concatenate(
        [vr0, vr1, vr2, vr3, jnp.zeros((4, D), jnp.float32)], axis=0)


def _main_body(nc, x_ref, wall_ref, biasp_ref, consts_ref, wvgg_ref,
               vrows_ref, bv_ref, wo_ref, bo_ref, nog_ref, nob_ref,
               out_ref, m_scr, t_scr, px_scr):
    D = D_MODEL
    c = pl.program_id(1)

    @pl.when(c == 0)
    def _():
        m_scr[...] = jnp.full((1, HP), NEG, jnp.float32)
        t_scr[...] = jnp.zeros((8, HP), jnp.float32)
        px_scr[...] = jnp.zeros((D, HP), jnp.float32)

    C = x_ref.shape[1]
    xh = x_ref[0].astype(jnp.bfloat16)
    wall = wall_ref[...]
    xs = _dot(xh, wall, ((1,), (0,)))
    x2s = _dot(xh * xh, wall, ((1,), (0,)))

    sT = jnp.transpose(xs[:, N_HEADS:N_HEADS + 8])
    s2T = jnp.transpose(x2s[:, N_HEADS:N_HEADS + 8])
    mu1, sg, sg2x, sgbx = sT[0:1], sT[1:2], sT[2:3], sT[3:4]
    mx2, sg2x2 = s2T[0:1], s2T[2:3]
    cr = consts_ref[...]
    mg, mb = cr[3:4, 0:1], cr[3:4, 1:2]
    mg2, mgb, mb2 = cr[3:4, 2:3], cr[3:4, 3:4], cr[3:4, 4:5]
    r1 = jax.lax.rsqrt(mx2 - mu1 * mu1 + EPS_LN)
    mu2 = r1 * (sg - mu1 * mg) + mb
    exn2 = (r1 * r1 * (sg2x2 - 2.0 * mu1 * sg2x + mu1 * mu1 * mg2)
            + 2.0 * r1 * (sgbx - mu1 * mgb) + mb2)
    r2 = jax.lax.rsqrt(exn2 - mu2 * mu2 + EPS_LN)
    a = r1 * r2
    coefT = jnp.concatenate(
        [r2, r2 * mu2, a * mu1, a, jnp.ones((1, C), jnp.float32),
         jnp.zeros((3, C), jnp.float32)], axis=0)

    crp = jnp.concatenate([cr[0:3], jnp.zeros((5, HP), jnp.float32)], axis=0)
    sub8 = jax.lax.broadcasted_iota(jnp.int32, (8, HP), 0)
    cra = jnp.where(sub8 == 3, 1.0, 0.0)
    pcorr = _dot(coefT, crp, ((0,), (0,)))
    afull = _dot(coefT, cra, ((0,), (0,)))
    s = afull * xs + pcorr + biasp_ref[...]

    cmax = jnp.max(s, axis=0, keepdims=True)
    m_old = m_scr[...]
    m_new = jnp.maximum(m_old, cmax)
    alpha = jnp.exp(m_old - m_new)
    p = jnp.exp(s - m_new)
    pb = p.astype(jnp.bfloat16)
    tup = _dot(coefT.astype(jnp.bfloat16), pb, ((1,), (0,)))
    t_new = t_scr[...] * alpha + tup
    px_new = px_scr[...] * alpha + _dot(xh, (p * afull).astype(jnp.bfloat16),
                                        ((0,), (0,)))
    m_scr[...] = m_new
    t_scr[...] = t_new
    px_scr[...] = px_new

    @pl.when(c == nc - 1)
    def _():
        d_new = t_new[4:5]
        inv = 1.0 / (d_new + 1e-9)
        inv16 = inv[:, :N_HEADS]
        pxf = (px_new[:, :N_HEADS] * inv16).astype(jnp.bfloat16)
        r2m = _dot(pxf, wvgg_ref[...], ((0,), (1,)))
        hsub = jax.lax.broadcasted_iota(jnp.int32, (N_HEADS, D), 0)
        hlane = jax.lax.broadcasted_iota(jnp.int32, (N_HEADS, D), 1)
        hmf = jnp.where(hsub == hlane // D_HEAD, 1.0, 0.0)
        merged = jnp.sum(r2m * hmf, axis=0, keepdims=True)
        ta4 = jnp.concatenate(
            [t_new[0:1, :N_HEADS] * inv16, t_new[1:2, :N_HEADS] * inv16,
             t_new[2:3, :N_HEADS] * inv16, d_new[:, :N_HEADS] * inv16],
            axis=0)
        sel = _dot(ta4, hmf, ((1,), (0,)))
        vr = vrows_ref[...]
        merged = (merged + sel[0:1] * vr[1:2] - sel[1:2] * vr[2:3]
                  - sel[2:3] * vr[3:4] + sel[3:4] * (vr[0:1] + bv_ref[...]))
        o = _dot(merged, wo_ref[...], ((1,), (1,))) + bo_ref[...]
        oc, ro = _rowstats(o)
        out_ref[0] = oc * ro * nog_ref[...] + nob_ref[...]


def kernel(x, query, norm_g, norm_b, nq_g, nq_b, nk_g, nk_b, nv_g, nv_b,
           no_g, no_b, Wq, bq, Wk, bk, Wv, bv, Wo, bo):
    B, S, D = x.shape
    C = min(4096, S)
    nc = S // C
    row = lambda v: v.reshape(1, D)

    wall, biasp, consts, wvgg, vrows = pl.pallas_call(
        _prologue_body,
        out_shape=[
            jax.ShapeDtypeStruct((D, HP), jnp.bfloat16),
            jax.ShapeDtypeStruct((1, HP), jnp.float32),
            jax.ShapeDtypeStruct((8, HP), jnp.float32),
            jax.ShapeDtypeStruct((D, D), jnp.bfloat16),
            jax.ShapeDtypeStruct((8, D), jnp.float32),
        ],
        name="pool_prologue",
    )(query.reshape(1, D), row(nq_g), row(nq_b), Wq, row(bq), Wk, row(bk),
      row(nk_g), row(nk_b), Wv, row(nv_g), row(nv_b), row(norm_g),
      row(norm_b))

    const = lambda shape: pl.BlockSpec(shape, lambda b, c: (0, 0))
    out = pl.pallas_call(
        functools.partial(_main_body, nc),
        out_shape=jax.ShapeDtypeStruct((B, 1, D), jnp.float32),
        grid=(B, nc),
        in_specs=[
            pl.BlockSpec((1, C, D), lambda b, c: (b, c, 0)),
            const((D, HP)), const((1, HP)), const((8, HP)),
            const((D, D)), const((8, D)),
            const((1, D)),
            const((D, D)), const((1, D)),
            const((1, D)), const((1, D)),
        ],
        out_specs=pl.BlockSpec((1, 1, D), lambda b, c: (b, 0, 0)),
        scratch_shapes=[
            pltpu.VMEM((1, HP), jnp.float32),
            pltpu.VMEM((8, HP), jnp.float32),
            pltpu.VMEM((D, HP), jnp.float32),
        ],
        compiler_params=pltpu.CompilerParams(
            dimension_semantics=("parallel", "arbitrary"),
            vmem_limit_bytes=100 * 1024 * 1024,
        ),
        name="pool_main",
    )(x, wall, biasp, consts, wvgg, vrows, row(bv),
      Wo, row(bo), row(no_g), row(no_b))
    return out.reshape(B, D)


# speedup vs baseline: 1.1850x; 1.1850x over previous
import functools

import jax
import jax.numpy as jnp
from jax.experimental import pallas as pl
from jax.experimental.pallas import tpu as pltpu

D_MODEL = 1024
N_HEADS = 16
D_HEAD = D_MODEL // N_HEADS
HP = 128
EPS_LN = 1e-5
NEG = -1e30
SCALE = 1.0 / (D_HEAD ** 0.5)


def _rowstats(x):
    mu = jnp.mean(x, axis=-1, keepdims=True)
    xc = x - mu
    var = jnp.mean(xc * xc, axis=-1, keepdims=True)
    return xc, jax.lax.rsqrt(var + EPS_LN)


def _dot(a, b, dims):
    return jax.lax.dot_general(a, b, (dims, ((), ())),
                               preferred_element_type=jnp.float32)


def _prologue_body(q_ref, nqg_ref, nqb_ref, wq_ref, bq_ref, wk_ref, bk_ref,
                   nkg_ref, nkb_ref, wv_ref, nvg_ref, nvb_ref, g_ref, b_ref,
                   wall_ref, biasp_ref, consts_ref, wvgg_ref, vrows_ref):
    D = D_MODEL
    g = g_ref[...]
    b = b_ref[...]
    ones = jnp.ones((1, D), jnp.float32)

    qc, qr = _rowstats(q_ref[...])
    qn = qc * qr * nqg_ref[...] + nqb_ref[...]
    qs = (_dot(qn, wq_ref[...], ((1,), (1,))) + bq_ref[...]) * SCALE

    lane = jax.lax.broadcasted_iota(jnp.int32, (HP, D), 1)
    sub = jax.lax.broadcasted_iota(jnp.int32, (HP, D), 0)
    ohT = jnp.where(sub == lane // D_HEAD, 1.0, 0.0)
    wsT = _dot(ohT * qs, wk_ref[...], ((1,), (0,)))

    part1 = _dot(qs * bk_ref[...], ohT, ((1,), (1,)))
    part2 = _dot(nkb_ref[...], wsT, ((1,), (1,)))
    hlane = jax.lax.broadcasted_iota(jnp.int32, (1, HP), 1)
    biasp_ref[...] = part1 + part2 + jnp.where(hlane < N_HEADS, 0.0, NEG)

    wsTk = wsT * nkg_ref[...]
    cb = _dot(b, wsTk, ((1,), (1,)))
    c1 = _dot(ones, wsTk, ((1,), (1,)))
    cg = _dot(g, wsTk, ((1,), (1,)))
    mg = jnp.mean(g, axis=-1, keepdims=True)
    mb = jnp.mean(b, axis=-1, keepdims=True)
    mg2 = jnp.mean(g * g, axis=-1, keepdims=True)
    mgb = jnp.mean(g * b, axis=-1, keepdims=True)
    mb2 = jnp.mean(b * b, axis=-1, keepdims=True)
    sl = hlane
    mrow = (jnp.where(sl == 0, mg, 0.0) + jnp.where(sl == 1, mb, 0.0)
            + jnp.where(sl == 2, mg2, 0.0) + jnp.where(sl == 3, mgb, 0.0)
            + jnp.where(sl == 4, mb2, 0.0))
    consts_ref[...] = jnp.concatenate(
        [cb, -c1, -cg, mrow, jnp.zeros((4, HP), jnp.float32)], axis=0)

    statT = (jnp.where(sub == 0, 1.0 / D, 0.0)
             + jnp.where(sub == 1, g / D, 0.0)
             + jnp.where(sub == 2, g * g / D, 0.0)
             + jnp.where(sub == 3, g * b / D, 0.0))
    wallT = jnp.concatenate([wsTk * g, statT], axis=0).astype(jnp.bfloat16)
    ii = jax.lax.broadcasted_iota(jnp.int32, (D, D), 0)
    jj = jax.lax.broadcasted_iota(jnp.int32, (D, D), 1)
    eyeb = jnp.where(ii == jj, 1.0, 0.0).astype(jnp.bfloat16)
    wall_ref[...] = _dot(eyeb, wallT, ((1,), (1,))).astype(jnp.bfloat16)

    wvg = wv_ref[...] * nvg_ref[...]
    wvgg_ref[...] = (wvg * g).astype(jnp.bfloat16)
    vr0 = _dot(nvb_ref[...], wv_ref[...], ((1,), (1,)))
    vr1 = _dot(b, wvg, ((1,), (1,)))
    vr2 = _dot(ones, wvg, ((1,), (1,)))
    vr3 = _dot(g, wvg, ((1,), (1,)))
    vrows_ref[...] = jnp.concatenate(
        [vr0, vr1, vr2, vr3, jnp.zeros((4, D), jnp.float32)], axis=0)


def _main_body(nc, x_ref, wall_ref, biasp_ref, consts_ref, wvgg_ref,
               vrows_ref, bv_ref, wo_ref, bo_ref, nog_ref, nob_ref,
               out_ref, m_scr, t_scr, px_scr):
    D = D_MODEL
    c = pl.program_id(1)

    @pl.when(c == 0)
    def _():
        m_scr[...] = jnp.full((1, HP), NEG, jnp.float32)
        t_scr[...] = jnp.zeros((8, HP), jnp.float32)
        px_scr[...] = jnp.zeros((D, HP), jnp.float32)

    C = x_ref.shape[1]
    xh = x_ref[0].astype(jnp.bfloat16)
    wall = wall_ref[...]
    xs = _dot(xh, wall, ((1,), (0,)))
    x2s = _dot(xh * xh, wall, ((1,), (0,)))

    sT = jnp.transpose(xs[:, HP:HP + 8])
    s2T = jnp.transpose(x2s[:, HP:HP + 8])
    mu1, sg, sg2x, sgbx = sT[0:1], sT[1:2], sT[2:3], sT[3:4]
    mx2, sg2x2 = s2T[0:1], s2T[2:3]
    cr = consts_ref[...]
    mg, mb = cr[3:4, 0:1], cr[3:4, 1:2]
    mg2, mgb, mb2 = cr[3:4, 2:3], cr[3:4, 3:4], cr[3:4, 4:5]
    r1 = jax.lax.rsqrt(mx2 - mu1 * mu1 + EPS_LN)
    mu2 = r1 * (sg - mu1 * mg) + mb
    exn2 = (r1 * r1 * (sg2x2 - 2.0 * mu1 * sg2x + mu1 * mu1 * mg2)
            + 2.0 * r1 * (sgbx - mu1 * mgb) + mb2)
    r2 = jax.lax.rsqrt(exn2 - mu2 * mu2 + EPS_LN)
    a = r1 * r2
    coefT = jnp.concatenate(
        [r2, r2 * mu2, a * mu1, a, jnp.ones((1, C), jnp.float32),
         jnp.zeros((3, C), jnp.float32)], axis=0)

    crp = jnp.concatenate([cr[0:3], jnp.zeros((5, HP), jnp.float32)], axis=0)
    sub8 = jax.lax.broadcasted_iota(jnp.int32, (8, HP), 0)
    cra = jnp.where(sub8 == 3, 1.0, 0.0)
    pcorr = _dot(coefT, crp, ((0,), (0,)))
    afull = _dot(coefT, cra, ((0,), (0,)))
    s = afull * xs[:, :HP] + pcorr + biasp_ref[...]

    cmax = jnp.max(s, axis=0, keepdims=True)
    m_old = m_scr[...]
    m_new = jnp.maximum(m_old, cmax)
    alpha = jnp.exp(m_old - m_new)
    p = jnp.exp(s - m_new)
    pb = p.astype(jnp.bfloat16)
    tup = _dot(coefT.astype(jnp.bfloat16), pb, ((1,), (0,)))
    t_new = t_scr[...] * alpha + tup
    px_new = px_scr[...] * alpha + _dot(xh, (p * afull).astype(jnp.bfloat16),
                                        ((0,), (0,)))
    m_scr[...] = m_new
    t_scr[...] = t_new
    px_scr[...] = px_new

    @pl.when(c == nc - 1)
    def _():
        d_new = t_new[4:5]
        inv = 1.0 / (d_new + 1e-9)
        inv16 = inv[:, :N_HEADS]
        pxf = (px_new[:, :N_HEADS] * inv16).astype(jnp.bfloat16)
        r2m = _dot(pxf, wvgg_ref[...], ((0,), (1,)))
        hsub = jax.lax.broadcasted_iota(jnp.int32, (N_HEADS, D), 0)
        hlane = jax.lax.broadcasted_iota(jnp.int32, (N_HEADS, D), 1)
        hmf = jnp.where(hsub == hlane // D_HEAD, 1.0, 0.0)
        merged = jnp.sum(r2m * hmf, axis=0, keepdims=True)
        ta4 = jnp.concatenate(
            [t_new[0:1, :N_HEADS] * inv16, t_new[1:2, :N_HEADS] * inv16,
             t_new[2:3, :N_HEADS] * inv16, d_new[:, :N_HEADS] * inv16],
            axis=0)
        sel = _dot(ta4, hmf, ((1,), (0,)))
        vr = vrows_ref[...]
        merged = (merged + sel[0:1] * vr[1:2] - sel[1:2] * vr[2:3]
                  - sel[2:3] * vr[3:4] + sel[3:4] * (vr[0:1] + bv_ref[...]))
        o = _dot(merged, wo_ref[...], ((1,), (1,))) + bo_ref[...]
        oc, ro = _rowstats(o)
        out_ref[0] = oc * ro * nog_ref[...] + nob_ref[...]


def kernel(x, query, norm_g, norm_b, nq_g, nq_b, nk_g, nk_b, nv_g, nv_b,
           no_g, no_b, Wq, bq, Wk, bk, Wv, bv, Wo, bo):
    B, S, D = x.shape
    C = min(4096, S)
    nc = S // C
    row = lambda v: v.reshape(1, D)

    wall, biasp, consts, wvgg, vrows = pl.pallas_call(
        _prologue_body,
        out_shape=[
            jax.ShapeDtypeStruct((D, 2 * HP), jnp.bfloat16),
            jax.ShapeDtypeStruct((1, HP), jnp.float32),
            jax.ShapeDtypeStruct((8, HP), jnp.float32),
            jax.ShapeDtypeStruct((D, D), jnp.bfloat16),
            jax.ShapeDtypeStruct((8, D), jnp.float32),
        ],
        name="pool_prologue",
    )(query.reshape(1, D), row(nq_g), row(nq_b), Wq, row(bq), Wk, row(bk),
      row(nk_g), row(nk_b), Wv, row(nv_g), row(nv_b), row(norm_g),
      row(norm_b))

    const = lambda shape: pl.BlockSpec(shape, lambda b, c: (0, 0))
    out = pl.pallas_call(
        functools.partial(_main_body, nc),
        out_shape=jax.ShapeDtypeStruct((B, 1, D), jnp.float32),
        grid=(B, nc),
        in_specs=[
            pl.BlockSpec((1, C, D), lambda b, c: (b, c, 0)),
            const((D, 2 * HP)), const((1, HP)), const((8, HP)),
            const((D, D)), const((8, D)),
            const((1, D)),
            const((D, D)), const((1, D)),
            const((1, D)), const((1, D)),
        ],
        out_specs=pl.BlockSpec((1, 1, D), lambda b, c: (b, 0, 0)),
        scratch_shapes=[
            pltpu.VMEM((1, HP), jnp.float32),
            pltpu.VMEM((8, HP), jnp.float32),
            pltpu.VMEM((D, HP), jnp.float32),
        ],
        compiler_params=pltpu.CompilerParams(
            dimension_semantics=("parallel", "arbitrary"),
            vmem_limit_bytes=100 * 1024 * 1024,
        ),
        name="pool_main",
    )(x, wall, biasp, consts, wvgg, vrows, row(bv),
      Wo, row(bo), row(no_g), row(no_b))
    return out.reshape(B, D)
